# baseline (speedup 1.0000x reference)
import functools

import jax
import jax.numpy as jnp
from jax.experimental import pallas as pl
from jax.experimental.pallas import tpu as pltpu

C = 256
N_HIDDEN = 5
BE = 1280


def _leaky(x):
    return jnp.where(x > 0, x, 0.01 * x)


def _dot(a, b):
    return jax.lax.dot_general(a.astype(jnp.bfloat16), b.astype(jnp.bfloat16),
                               (((1,), (0,)), ((), ())),
                               preferred_element_type=jnp.float32)


def _mlp_tail(g, w_ref, b_ref):
    h = _leaky(g)
    for i in range(N_HIDDEN - 1):
        h = _leaky(_dot(h, w_ref[i]) + b_ref[i : i + 1, :])
    return _dot(h, w_ref[N_HIDDEN - 1]) + b_ref[N_HIDDEN - 1 : N_HIDDEN, :]


def _mlp5_kernel(g_ref, w_ref, b_ref, out_ref):
    out_ref[...] = _mlp_tail(g_ref[...], w_ref, b_ref)


def _mlp5_pair_kernel(glo_ref, ghi_ref, ew_ref, w_ref, b_ref, out_ref):
    m = _mlp_tail(glo_ref[...], w_ref, b_ref) + _mlp_tail(ghi_ref[...], w_ref, b_ref)
    out_ref[...] = m * ew_ref[...]


def _e2_head_kernel(glo_ref, ghi_ref, ef1g_ref, eff_ref, ew_ref,
                    wm_ref, bm_ref, we_ref, be_ref, wh_ref, bh_ref,
                    out_ref):
    ew = ew_ref[...]
    m = _mlp_tail(glo_ref[...], wm_ref, bm_ref) + _mlp_tail(ghi_ref[...], wm_ref, bm_ref)
    ef2pre = m * ew
    z = _dot(ef2pre, we_ref[0]) + ef1g_ref[...] + be_ref[0:1, :]
    h = _leaky(z)
    for i in range(1, N_HIDDEN):
        h = _leaky(_dot(h, we_ref[i]) + be_ref[i : i + 1, :])
    ef2 = _leaky(_dot(h, we_ref[N_HIDDEN]) + be_ref[N_HIDDEN : N_HIDDEN + 1, :])
    e1 = _dot(ef2, wh_ref[...]) + eff_ref[...] + bh_ref[0:1, :]
    s0 = jnp.sum(e1 * bh_ref[1:2, :], axis=1, keepdims=True) + bh_ref[3:4, 0:1]
    s1 = jnp.sum(e1 * bh_ref[2:3, :], axis=1, keepdims=True) + bh_ref[3:4, 1:2]
    s0 = jax.nn.sigmoid(s0)
    s1 = jax.nn.sigmoid(s1)
    mx = jnp.maximum(s0, s1)
    z0 = jnp.exp(s0 - mx)
    z1 = jnp.exp(s1 - mx)
    tot = z0 + z1
    out_ref[...] = jnp.concatenate([z0 / tot, z1 / tot], axis=1)


def _full_spec(shape):
    return pl.BlockSpec(shape, lambda i: tuple(0 for _ in shape))


def _mlp5(g, wh, bh):
    n = g.shape[0]
    grid = (n // BE,)
    return pl.pallas_call(
        _mlp5_kernel,
        grid=grid,
        in_specs=[
            pl.BlockSpec((BE, C), lambda i: (i, 0)),
            _full_spec(wh.shape),
            _full_spec(bh.shape),
        ],
        out_specs=pl.BlockSpec((BE, C), lambda i: (i, 0)),
        out_shape=jax.ShapeDtypeStruct((n, C), jnp.float32),
    )(g, wh, bh)


def _mlp5_pair(glo, ghi, ew, wh, bh):
    n = glo.shape[0]
    grid = (n // BE,)
    return pl.pallas_call(
        _mlp5_pair_kernel,
        grid=grid,
        in_specs=[
            pl.BlockSpec((BE, C), lambda i: (i, 0)),
            pl.BlockSpec((BE, C), lambda i: (i, 0)),
            pl.BlockSpec((BE, 1), lambda i: (i, 0)),
            _full_spec(wh.shape),
            _full_spec(bh.shape),
        ],
        out_specs=pl.BlockSpec((BE, C), lambda i: (i, 0)),
        out_shape=jax.ShapeDtypeStruct((n, C), jnp.float32),
    )(glo, ghi, ew, wh, bh)


def _e2_head(glo, ghi, ef1g, eff, ew, wm, bm, we, be, wh, bh):
    n = glo.shape[0]
    grid = (n // BE,)
    return pl.pallas_call(
        _e2_head_kernel,
        grid=grid,
        in_specs=[
            pl.BlockSpec((BE, C), lambda i: (i, 0)),
            pl.BlockSpec((BE, C), lambda i: (i, 0)),
            pl.BlockSpec((BE, C), lambda i: (i, 0)),
            pl.BlockSpec((BE, C), lambda i: (i, 0)),
            pl.BlockSpec((BE, 1), lambda i: (i, 0)),
            _full_spec(wm.shape),
            _full_spec(bm.shape),
            _full_spec(we.shape),
            _full_spec(be.shape),
            _full_spec(wh.shape),
            _full_spec(bh.shape),
        ],
        out_specs=pl.BlockSpec((BE, 2), lambda i: (i, 0)),
        out_shape=jax.ShapeDtypeStruct((n, 2), jnp.float32),
    )(glo, ghi, ef1g, eff, ew, wm, bm, we, be, wh, bh)


def _stack_tail(pars):
    wh = jnp.stack([w for (w, b) in pars[1:]])
    bh = jnp.stack([b for (w, b) in pars[1:]])
    return wh, bh


def kernel(node_features, edge_features_1d, edge_index, angles, edge_weights, params):
    src = edge_index[0].astype(jnp.int32)
    dst = edge_index[1].astype(jnp.int32)
    n_nodes = node_features.shape[0]
    e_und = edge_weights.shape[0]
    src_lo, src_hi = src[:e_und], src[e_und:]
    dst_lo, dst_hi = dst[:e_und], dst[e_und:]

    cnt = jax.ops.segment_sum(jnp.ones_like(dst, jnp.float32), dst, num_segments=n_nodes)
    inv = 1.0 / jnp.maximum(cnt, 1.0)

    def node_stage(x, pars):
        w0, b0 = pars[0]
        ps = x @ w0[:C] + b0[None, :]
        pd = x @ w0[C : 2 * C]
        g = ps[src] + pd[dst] + angles * w0[2 * C][None, :]
        wh, bh = _stack_tail(pars)
        m = _mlp5(g, wh, bh)
        s = jax.ops.segment_sum(m, dst, num_segments=n_nodes)
        return _leaky(s * inv[:, None])

    x1 = node_stage(node_features, params['node_conv1_msg'])

    v0, c0 = params['edge_conv1_msg'][0]
    qs = x1 @ v0[:C] + c0[None, :]
    qd = x1 @ v0[C:]
    glo = qs[src_lo] + qd[dst_lo]
    ghi = qs[src_hi] + qd[dst_hi]
    wh1, bh1 = _stack_tail(params['edge_conv1_msg'])
    ef1 = _leaky(_mlp5_pair(glo, ghi, edge_weights[:, None], wh1, bh1))

    x2 = node_stage(x1, params['node_conv2_msg'])

    u0, d0 = params['edge_conv2_msg'][0]
    rs = x2 @ u0[:C] + d0[None, :]
    rd = x2 @ u0[C:]
    g2lo = rs[src_lo] + rd[dst_lo]
    g2hi = rs[src_hi] + rd[dst_hi]
    wm, bm = _stack_tail(params['edge_conv2_msg'])

    epars = params['edge_conv2_edge']
    we = jnp.stack([epars[0][0][:C]] + [w for (w, b) in epars[1:]])
    be = jnp.stack([b for (w, b) in epars])
    ef1g = ef1 @ epars[0][0][C:]

    (wl1, bl1) = params['out_lcf1'][0]
    (wl2, bl2) = params['out_lcf2'][0]
    eff = edge_features_1d @ wl1[C : C + 16] + edge_weights[:, None] * wl1[C + 16][None, :]
    wh_head = wl1[:C]
    bh_head = jnp.stack([
        bl1,
        wl2[:, 0],
        wl2[:, 1],
        jnp.concatenate([bl2, jnp.zeros((C - 2,), jnp.float32)]),
    ])

    return _e2_head(g2lo, g2hi, ef1g, eff, edge_weights[:, None],
                    wm, bm, we, be, wh_head, bh_head)


# speedup vs baseline: 1.2175x; 1.2175x over previous
import functools

import jax
import jax.numpy as jnp
from jax import lax
from jax.experimental import pallas as pl
from jax.experimental.pallas import tpu as pltpu
from jax.experimental.pallas import tpu_sc as plsc

C = 256
N_HIDDEN = 5
BE = 1280
KCH = 104
CW = 100
NW = 32
PADH = 16 * KCH * CW
PADT = 2 * PADH


def _leaky(x):
    return jnp.where(x > 0, x, 0.01 * x)


def _dot(a, b):
    return jax.lax.dot_general(a.astype(jnp.bfloat16), b.astype(jnp.bfloat16),
                               (((1,), (0,)), ((), ())),
                               preferred_element_type=jnp.float32)


def _sc_gather2(taba, tabb, idxa, idxb):
    mesh = plsc.VectorSubcoreMesh(core_axis_name="c", subcore_axis_name="s")

    @functools.partial(
        pl.kernel,
        mesh=mesh,
        out_type=(jax.ShapeDtypeStruct((PADT, C), jnp.float32),
                  jax.ShapeDtypeStruct((PADT, C), jnp.float32)),
        scratch_types=(
            [pltpu.VMEM((KCH, C), jnp.float32)] * 4
            + [pltpu.VMEM((KCH,), jnp.int32)] * 8
            + [pltpu.SemaphoreType.DMA] * 16
        ),
    )
    def k(ta_h, tb_h, ia_h, ib_h, oa_h, ob_h,
          ra0, ra1, rb0, rb1,
          ja0, ja1, ja2, ja3, jb0, jb1, jb2, jb3,
          sia0, sia1, sia2, sia3, sib0, sib1, sib2, sib3,
          sga0, sga1, sgb0, sgb1, ssa0, ssa1, ssb0, ssb1):
        wid = lax.axis_index("s") * 2 + lax.axis_index("c")
        base = wid * CW

        ra = (ra0, ra1)
        rb = (rb0, rb1)
        ja = (ja0, ja1, ja2, ja3)
        jb = (jb0, jb1, jb2, jb3)
        sia = (sia0, sia1, sia2, sia3)
        sib = (sib0, sib1, sib2, sib3)
        sga = (sga0, sga1)
        sgb = (sgb0, sgb1)
        ssa = (ssa0, ssa1)
        ssb = (ssb0, ssb1)

        def fire_idx(j, q):
            off = (base + j) * KCH
            pltpu.async_copy(ia_h.at[pl.ds(off, KCH)], ja[q], sia[q])
            pltpu.async_copy(ib_h.at[pl.ds(off, KCH)], jb[q], sib[q])

        def gath(q, p):
            pltpu.make_async_copy(ia_h.at[pl.ds(0, KCH)], ja[q], sia[q]).wait()
            pltpu.make_async_copy(ib_h.at[pl.ds(0, KCH)], jb[q], sib[q]).wait()
            pltpu.async_copy(ta_h.at[ja[q]], ra[p], sga[p])
            pltpu.async_copy(tb_h.at[jb[q]], rb[p], sgb[p])

        def stor(j, q, p):
            off = (base + j) * KCH
            pltpu.make_async_copy(ta_h.at[ja[q]], ra[p], sga[p]).wait()
            pltpu.make_async_copy(tb_h.at[jb[q]], rb[p], sgb[p]).wait()
            pltpu.async_copy(ra[p], oa_h.at[pl.ds(off, KCH)], ssa[p])
            pltpu.async_copy(rb[p], ob_h.at[pl.ds(off, KCH)], ssb[p])

        def wait_stor(p):
            pltpu.make_async_copy(ra[p], oa_h.at[pl.ds(0, KCH)], ssa[p]).wait()
            pltpu.make_async_copy(rb[p], ob_h.at[pl.ds(0, KCH)], ssb[p]).wait()

        fire_idx(0, 0)
        fire_idx(1, 1)
        fire_idx(2, 2)
        fire_idx(3, 3)

        def body(jj, carry):
            for u in range(4):
                j = 4 * jj + u
                q = u
                p = u % 2

                if u >= 2:
                    wait_stor(p)
                else:
                    @pl.when(jj > 0)
                    def _():
                        wait_stor(p)

                gath(q, p)

                if u >= 1:
                    qp = u - 1
                    pp = (u - 1) % 2
                    stor(j - 1, qp, pp)
                    fire_j = j + 3

                    @pl.when(fire_j < CW)
                    def _():
                        fire_idx(fire_j, qp)
                else:
                    @pl.when(jj > 0)
                    def _():
                        stor(j - 1, 3, 1)

                        @pl.when(j + 3 < CW)
                        def _():
                            fire_idx(j + 3, 3)

            return carry

        lax.fori_loop(0, CW // 4, body, 0)
        stor(CW - 1, 3, 1)
        wait_stor(0)
        wait_stor(1)

    return k(taba, tabb, idxa, idxb)


def _mlp_tail(g, w_ref, b_ref):
    h = _leaky(g)
    for i in range(N_HIDDEN - 1):
        h = _leaky(_dot(h, w_ref[i]) + b_ref[i : i + 1, :])
    return _dot(h, w_ref[N_HIDDEN - 1]) + b_ref[N_HIDDEN - 1 : N_HIDDEN, :]


def _mlp5n_kernel(ga_ref, gb_ref, ang_ref, w0a_ref, w_ref, b_ref, out_ref):
    g = ga_ref[...] + gb_ref[...] + ang_ref[...] * w0a_ref[...]
    out_ref[...] = _mlp_tail(g, w_ref, b_ref)


def _mlp5_pair_kernel(galo_ref, gblo_ref, gahi_ref, gbhi_ref, ew_ref,
                      w_ref, b_ref, out_ref):
    mlo = _mlp_tail(galo_ref[...] + gblo_ref[...], w_ref, b_ref)
    mhi = _mlp_tail(gahi_ref[...] + gbhi_ref[...], w_ref, b_ref)
    out_ref[...] = (mlo + mhi) * ew_ref[...]


def _e2_head_kernel(galo_ref, gblo_ref, gahi_ref, gbhi_ref, ef1g_ref, eff_ref,
                    ew_ref, wm_ref, bm_ref, we_ref, be_ref, wh_ref, bh_ref,
                    out_ref):
    ew = ew_ref[...]
    m = (_mlp_tail(galo_ref[...] + gblo_ref[...], wm_ref, bm_ref)
         + _mlp_tail(gahi_ref[...] + gbhi_ref[...], wm_ref, bm_ref))
    ef2pre = m * ew
    z = _dot(ef2pre, we_ref[0]) + ef1g_ref[...] + be_ref[0:1, :]
    h = _leaky(z)
    for i in range(1, N_HIDDEN):
        h = _leaky(_dot(h, we_ref[i]) + be_ref[i : i + 1, :])
    ef2 = _leaky(_dot(h, we_ref[N_HIDDEN]) + be_ref[N_HIDDEN : N_HIDDEN + 1, :])
    e1 = _dot(ef2, wh_ref[...]) + eff_ref[...] + bh_ref[0:1, :]
    s0 = jnp.sum(e1 * bh_ref[1:2, :], axis=1, keepdims=True) + bh_ref[3:4, 0:1]
    s1 = jnp.sum(e1 * bh_ref[2:3, :], axis=1, keepdims=True) + bh_ref[3:4, 1:2]
    s0 = jax.nn.sigmoid(s0)
    s1 = jax.nn.sigmoid(s1)
    mx = jnp.maximum(s0, s1)
    z0 = jnp.exp(s0 - mx)
    z1 = jnp.exp(s1 - mx)
    tot = z0 + z1
    out_ref[...] = jnp.concatenate([z0 / tot, z1 / tot], axis=1)


def _full_spec(shape):
    return pl.BlockSpec(shape, lambda i: tuple(0 for _ in shape))


def _row_spec(cols=C):
    return pl.BlockSpec((BE, cols), lambda i: (i, 0))


def _hi_spec(cols=C):
    off = PADH // BE
    return pl.BlockSpec((BE, cols), lambda i: (i + off, 0))


def _mlp5n(ga, gb, ang, w0a, wh, bh):
    grid = (PADT // BE,)
    return pl.pallas_call(
        _mlp5n_kernel,
        grid=grid,
        in_specs=[_row_spec(), _row_spec(), _row_spec(1),
                  _full_spec(w0a.shape), _full_spec(wh.shape), _full_spec(bh.shape)],
        out_specs=_row_spec(),
        out_shape=jax.ShapeDtypeStruct((PADT, C), jnp.float32),
    )(ga, gb, ang, w0a, wh, bh)


def _mlp5_pair(ga, gb, ew, wh, bh):
    grid = (PADH // BE,)
    return pl.pallas_call(
        _mlp5_pair_kernel,
        grid=grid,
        in_specs=[_row_spec(), _row_spec(), _hi_spec(), _hi_spec(),
                  pl.BlockSpec((BE, 1), lambda i: (i, 0)),
                  _full_spec(wh.shape), _full_spec(bh.shape)],
        out_specs=_row_spec(),
        out_shape=jax.ShapeDtypeStruct((PADH, C), jnp.float32),
    )(ga, gb, ga, gb, ew, wh, bh)


def _e2_head(ga, gb, ef1g, eff, ew, wm, bm, we, be, wh, bh):
    grid = (PADH // BE,)
    return pl.pallas_call(
        _e2_head_kernel,
        grid=grid,
        in_specs=[_row_spec(), _row_spec(), _hi_spec(), _hi_spec(),
                  _row_spec(), _row_spec(),
                  pl.BlockSpec((BE, 1), lambda i: (i, 0)),
                  _full_spec(wm.shape), _full_spec(bm.shape),
                  _full_spec(we.shape), _full_spec(be.shape),
                  _full_spec(wh.shape), _full_spec(bh.shape)],
        out_specs=pl.BlockSpec((BE, 2), lambda i: (i, 0)),
        out_shape=jax.ShapeDtypeStruct((PADH, 2), jnp.float32),
    )(ga, gb, ga, gb, ef1g, eff, ew, wm, bm, we, be, wh, bh)


def _stack_tail(pars):
    wh = jnp.stack([w for (w, b) in pars[1:]])
    bh = jnp.stack([b for (w, b) in pars[1:]])
    return wh, bh


def _pad_half(a, n):
    pad = jnp.zeros((PADH - n,) + a.shape[1:], a.dtype)
    return jnp.concatenate([a[:n], pad, a[n:], pad], axis=0)


def kernel(node_features, edge_features_1d, edge_index, angles, edge_weights, params):
    src = edge_index[0].astype(jnp.int32)
    dst = edge_index[1].astype(jnp.int32)
    n_nodes = node_features.shape[0]
    e_und = edge_weights.shape[0]

    src_pad = _pad_half(src, e_und)
    dst_pad = _pad_half(dst, e_und)
    half_mask = jnp.concatenate([
        jnp.zeros((e_und,), jnp.int32),
        jnp.ones((PADH - e_und,), jnp.int32),
    ])
    pad_mask = jnp.concatenate([half_mask, half_mask])
    dst_scat = jnp.where(pad_mask == 1, n_nodes, dst_pad)

    ang_pad = _pad_half(angles, e_und)
    ew_pad = jnp.concatenate(
        [edge_weights[:, None],
         jnp.zeros((PADH - e_und, 1), jnp.float32)], axis=0)

    cnt = jax.ops.segment_sum(jnp.ones((PADT,), jnp.float32), dst_scat,
                              num_segments=n_nodes + 1)[:n_nodes]
    inv = 1.0 / jnp.maximum(cnt, 1.0)

    def node_stage(x, pars):
        w0, b0 = pars[0]
        ta = x @ w0[:C] + b0[None, :]
        tb = x @ w0[C : 2 * C]
        ga, gb = _sc_gather2(ta, tb, src_pad, dst_pad)
        wh, bh = _stack_tail(pars)
        m = _mlp5n(ga, gb, ang_pad, w0[2 * C][None, :], wh, bh)
        s = jax.ops.segment_sum(m, dst_scat, num_segments=n_nodes + 1)[:n_nodes]
        return _leaky(s * inv[:, None])

    x1 = node_stage(node_features, params['node_conv1_msg'])

    v0, c0 = params['edge_conv1_msg'][0]
    ta = x1 @ v0[:C] + c0[None, :]
    tb = x1 @ v0[C:]
    ga1, gb1 = _sc_gather2(ta, tb, src_pad, dst_pad)
    wh1, bh1 = _stack_tail(params['edge_conv1_msg'])
    ef1 = _leaky(_mlp5_pair(ga1, gb1, ew_pad, wh1, bh1))

    x2 = node_stage(x1, params['node_conv2_msg'])

    u0, d0 = params['edge_conv2_msg'][0]
    ta2 = x2 @ u0[:C] + d0[None, :]
    tb2 = x2 @ u0[C:]
    ga2, gb2 = _sc_gather2(ta2, tb2, src_pad, dst_pad)
    wm, bm = _stack_tail(params['edge_conv2_msg'])

    epars = params['edge_conv2_edge']
    we = jnp.stack([epars[0][0][:C]] + [w for (w, b) in epars[1:]])
    be = jnp.stack([b for (w, b) in epars])
    ef1g = ef1 @ epars[0][0][C:]

    (wl1, bl1) = params['out_lcf1'][0]
    (wl2, bl2) = params['out_lcf2'][0]
    ef_pad = jnp.concatenate(
        [edge_features_1d,
         jnp.zeros((PADH - e_und, edge_features_1d.shape[1]), jnp.float32)], axis=0)
    eff = ef_pad @ wl1[C : C + 16] + ew_pad * wl1[C + 16][None, :]
    wh_head = wl1[:C]
    bh_head = jnp.stack([
        bl1,
        wl2[:, 0],
        wl2[:, 1],
        jnp.concatenate([bl2, jnp.zeros((C - 2,), jnp.float32)]),
    ])

    out = _e2_head(ga2, gb2, ef1g, eff, ew_pad,
                   wm, bm, we, be, wh_head, bh_head)
    return out[:e_und]


# speedup vs baseline: 1.2280x; 1.0087x over previous
import functools

import jax
import jax.numpy as jnp
from jax import lax
from jax.experimental import pallas as pl
from jax.experimental.pallas import tpu as pltpu
from jax.experimental.pallas import tpu_sc as plsc

C = 256
N_HIDDEN = 5
BE = 1280
KCH = 104
CW = 100
NW = 32
PADH = 16 * KCH * CW
PADT = 2 * PADH


def _leaky(x):
    return jnp.maximum(x, 0.01 * x)


def _dot(a, b, out_dtype=jnp.float32):
    return jax.lax.dot_general(a.astype(jnp.bfloat16), b.astype(jnp.bfloat16),
                               (((1,), (0,)), ((), ())),
                               preferred_element_type=out_dtype)


def _sc_gather2(taba, tabb, idxa, idxb):
    mesh = plsc.VectorSubcoreMesh(core_axis_name="c", subcore_axis_name="s")

    @functools.partial(
        pl.kernel,
        mesh=mesh,
        out_type=(jax.ShapeDtypeStruct((PADT, C), jnp.float32),
                  jax.ShapeDtypeStruct((PADT, C), jnp.float32)),
        scratch_types=(
            [pltpu.VMEM((KCH, C), jnp.float32)] * 4
            + [pltpu.VMEM((KCH,), jnp.int32)] * 8
            + [pltpu.SemaphoreType.DMA] * 16
        ),
    )
    def k(ta_h, tb_h, ia_h, ib_h, oa_h, ob_h,
          ra0, ra1, rb0, rb1,
          ja0, ja1, ja2, ja3, jb0, jb1, jb2, jb3,
          sia0, sia1, sia2, sia3, sib0, sib1, sib2, sib3,
          sga0, sga1, sgb0, sgb1, ssa0, ssa1, ssb0, ssb1):
        wid = lax.axis_index("s") * 2 + lax.axis_index("c")
        base = wid * CW

        ra = (ra0, ra1)
        rb = (rb0, rb1)
        ja = (ja0, ja1, ja2, ja3)
        jb = (jb0, jb1, jb2, jb3)
        sia = (sia0, sia1, sia2, sia3)
        sib = (sib0, sib1, sib2, sib3)
        sga = (sga0, sga1)
        sgb = (sgb0, sgb1)
        ssa = (ssa0, ssa1)
        ssb = (ssb0, ssb1)

        def fire_idx(j, q):
            off = (base + j) * KCH
            pltpu.async_copy(ia_h.at[pl.ds(off, KCH)], ja[q], sia[q])
            pltpu.async_copy(ib_h.at[pl.ds(off, KCH)], jb[q], sib[q])

        def gath(q, p):
            pltpu.make_async_copy(ia_h.at[pl.ds(0, KCH)], ja[q], sia[q]).wait()
            pltpu.make_async_copy(ib_h.at[pl.ds(0, KCH)], jb[q], sib[q]).wait()
            pltpu.async_copy(ta_h.at[ja[q]], ra[p], sga[p])
            pltpu.async_copy(tb_h.at[jb[q]], rb[p], sgb[p])

        def stor(j, q, p):
            off = (base + j) * KCH
            pltpu.make_async_copy(ta_h.at[ja[q]], ra[p], sga[p]).wait()
            pltpu.make_async_copy(tb_h.at[jb[q]], rb[p], sgb[p]).wait()
            pltpu.async_copy(ra[p], oa_h.at[pl.ds(off, KCH)], ssa[p])
            pltpu.async_copy(rb[p], ob_h.at[pl.ds(off, KCH)], ssb[p])

        def wait_stor(p):
            pltpu.make_async_copy(ra[p], oa_h.at[pl.ds(0, KCH)], ssa[p]).wait()
            pltpu.make_async_copy(rb[p], ob_h.at[pl.ds(0, KCH)], ssb[p]).wait()

        fire_idx(0, 0)
        fire_idx(1, 1)
        fire_idx(2, 2)
        fire_idx(3, 3)

        def body(jj, carry):
            for u in range(4):
                j = 4 * jj + u
                q = u
                p = u % 2

                if u >= 2:
                    wait_stor(p)
                else:
                    @pl.when(jj > 0)
                    def _():
                        wait_stor(p)

                gath(q, p)

                if u >= 1:
                    qp = u - 1
                    pp = (u - 1) % 2
                    stor(j - 1, qp, pp)
                    fire_j = j + 3

                    @pl.when(fire_j < CW)
                    def _():
                        fire_idx(fire_j, qp)
                else:
                    @pl.when(jj > 0)
                    def _():
                        stor(j - 1, 3, 1)

                        @pl.when(j + 3 < CW)
                        def _():
                            fire_idx(j + 3, 3)

            return carry

        lax.fori_loop(0, CW // 4, body, 0)
        stor(CW - 1, 3, 1)
        wait_stor(0)
        wait_stor(1)

    return k(taba, tabb, idxa, idxb)


def _mlp_tail(g, w_ref, b_ref):
    h = _leaky(g).astype(jnp.bfloat16)
    for i in range(N_HIDDEN - 1):
        z = _dot(h, w_ref[i]).astype(jnp.bfloat16)
        h = _leaky(z + b_ref[i : i + 1, :].astype(jnp.bfloat16))
    return _dot(h, w_ref[N_HIDDEN - 1]) + b_ref[N_HIDDEN - 1 : N_HIDDEN, :]


def _mlp5n_kernel(ga_ref, gb_ref, ang_ref, w0a_ref, w_ref, b_ref, out_ref):
    g = ga_ref[...] + gb_ref[...] + ang_ref[...] * w0a_ref[...]
    out_ref[...] = _mlp_tail(g, w_ref, b_ref)


def _mlp5_pair_kernel(galo_ref, gblo_ref, gahi_ref, gbhi_ref, ew_ref,
                      w_ref, b_ref, out_ref):
    mlo = _mlp_tail(galo_ref[...] + gblo_ref[...], w_ref, b_ref)
    mhi = _mlp_tail(gahi_ref[...] + gbhi_ref[...], w_ref, b_ref)
    out_ref[...] = (mlo + mhi) * ew_ref[...]


def _e2_head_kernel(galo_ref, gblo_ref, gahi_ref, gbhi_ref, ef1g_ref, eff_ref,
                    ew_ref, wm_ref, bm_ref, we_ref, be_ref, wh_ref, bh_ref,
                    out_ref):
    ew = ew_ref[...]
    m = (_mlp_tail(galo_ref[...] + gblo_ref[...], wm_ref, bm_ref)
         + _mlp_tail(gahi_ref[...] + gbhi_ref[...], wm_ref, bm_ref))
    ef2pre = m * ew
    z = _dot(ef2pre, we_ref[0]) + ef1g_ref[...] + be_ref[0:1, :]
    h = _leaky(z).astype(jnp.bfloat16)
    for i in range(1, N_HIDDEN):
        zz = _dot(h, we_ref[i]).astype(jnp.bfloat16)
        h = _leaky(zz + be_ref[i : i + 1, :].astype(jnp.bfloat16))
    ef2 = _leaky(_dot(h, we_ref[N_HIDDEN]) + be_ref[N_HIDDEN : N_HIDDEN + 1, :])
    e1 = _dot(ef2, wh_ref[...]) + eff_ref[...] + bh_ref[0:1, :]
    s0 = jnp.sum(e1 * bh_ref[1:2, :], axis=1, keepdims=True) + bh_ref[3:4, 0:1]
    s1 = jnp.sum(e1 * bh_ref[2:3, :], axis=1, keepdims=True) + bh_ref[3:4, 1:2]
    s0 = jax.nn.sigmoid(s0)
    s1 = jax.nn.sigmoid(s1)
    mx = jnp.maximum(s0, s1)
    z0 = jnp.exp(s0 - mx)
    z1 = jnp.exp(s1 - mx)
    tot = z0 + z1
    out_ref[...] = jnp.concatenate([z0 / tot, z1 / tot], axis=1)


def _full_spec(shape):
    return pl.BlockSpec(shape, lambda i: tuple(0 for _ in shape))


def _row_spec(cols=C):
    return pl.BlockSpec((BE, cols), lambda i: (i, 0))


def _hi_spec(cols=C):
    off = PADH // BE
    return pl.BlockSpec((BE, cols), lambda i: (i + off, 0))


def _mlp5n(ga, gb, ang, w0a, wh, bh):
    grid = (PADT // BE,)
    return pl.pallas_call(
        _mlp5n_kernel,
        grid=grid,
        in_specs=[_row_spec(), _row_spec(), _row_spec(1),
                  _full_spec(w0a.shape), _full_spec(wh.shape), _full_spec(bh.shape)],
        out_specs=_row_spec(),
        out_shape=jax.ShapeDtypeStruct((PADT, C), jnp.float32),
    )(ga, gb, ang, w0a, wh, bh)


def _mlp5_pair(ga, gb, ew, wh, bh):
    grid = (PADH // BE,)
    return pl.pallas_call(
        _mlp5_pair_kernel,
        grid=grid,
        in_specs=[_row_spec(), _row_spec(), _hi_spec(), _hi_spec(),
                  pl.BlockSpec((BE, 1), lambda i: (i, 0)),
                  _full_spec(wh.shape), _full_spec(bh.shape)],
        out_specs=_row_spec(),
        out_shape=jax.ShapeDtypeStruct((PADH, C), jnp.float32),
    )(ga, gb, ga, gb, ew, wh, bh)


def _e2_head(ga, gb, ef1g, eff, ew, wm, bm, we, be, wh, bh):
    grid = (PADH // BE,)
    return pl.pallas_call(
        _e2_head_kernel,
        grid=grid,
        in_specs=[_row_spec(), _row_spec(), _hi_spec(), _hi_spec(),
                  _row_spec(), _row_spec(),
                  pl.BlockSpec((BE, 1), lambda i: (i, 0)),
                  _full_spec(wm.shape), _full_spec(bm.shape),
                  _full_spec(we.shape), _full_spec(be.shape),
                  _full_spec(wh.shape), _full_spec(bh.shape)],
        out_specs=pl.BlockSpec((BE, 2), lambda i: (i, 0)),
        out_shape=jax.ShapeDtypeStruct((PADH, 2), jnp.float32),
    )(ga, gb, ga, gb, ef1g, eff, ew, wm, bm, we, be, wh, bh)


def _stack_tail(pars):
    wh = jnp.stack([w for (w, b) in pars[1:]])
    bh = jnp.stack([b for (w, b) in pars[1:]])
    return wh, bh


def _pad_half(a, n):
    pad = jnp.zeros((PADH - n,) + a.shape[1:], a.dtype)
    return jnp.concatenate([a[:n], pad, a[n:], pad], axis=0)


def kernel(node_features, edge_features_1d, edge_index, angles, edge_weights, params):
    src = edge_index[0].astype(jnp.int32)
    dst = edge_index[1].astype(jnp.int32)
    n_nodes = node_features.shape[0]
    e_und = edge_weights.shape[0]

    src_pad = _pad_half(src, e_und)
    dst_pad = _pad_half(dst, e_und)
    half_mask = jnp.concatenate([
        jnp.zeros((e_und,), jnp.int32),
        jnp.ones((PADH - e_und,), jnp.int32),
    ])
    pad_mask = jnp.concatenate([half_mask, half_mask])
    dst_scat = jnp.where(pad_mask == 1, n_nodes, dst_pad)

    ang_pad = _pad_half(angles, e_und)
    ew_pad = jnp.concatenate(
        [edge_weights[:, None],
         jnp.zeros((PADH - e_und, 1), jnp.float32)], axis=0)

    cnt = jax.ops.segment_sum(jnp.ones((PADT,), jnp.float32), dst_scat,
                              num_segments=n_nodes + 1)[:n_nodes]
    inv = 1.0 / jnp.maximum(cnt, 1.0)

    def node_stage(x, pars):
        w0, b0 = pars[0]
        ta = x @ w0[:C] + b0[None, :]
        tb = x @ w0[C : 2 * C]
        ga, gb = _sc_gather2(ta, tb, src_pad, dst_pad)
        wh, bh = _stack_tail(pars)
        m = _mlp5n(ga, gb, ang_pad, w0[2 * C][None, :], wh, bh)
        s = jax.ops.segment_sum(m, dst_scat, num_segments=n_nodes + 1)[:n_nodes]
        return _leaky(s * inv[:, None])

    x1 = node_stage(node_features, params['node_conv1_msg'])

    v0, c0 = params['edge_conv1_msg'][0]
    ta = x1 @ v0[:C] + c0[None, :]
    tb = x1 @ v0[C:]
    ga1, gb1 = _sc_gather2(ta, tb, src_pad, dst_pad)
    wh1, bh1 = _stack_tail(params['edge_conv1_msg'])
    ef1 = _leaky(_mlp5_pair(ga1, gb1, ew_pad, wh1, bh1))

    x2 = node_stage(x1, params['node_conv2_msg'])

    u0, d0 = params['edge_conv2_msg'][0]
    ta2 = x2 @ u0[:C] + d0[None, :]
    tb2 = x2 @ u0[C:]
    ga2, gb2 = _sc_gather2(ta2, tb2, src_pad, dst_pad)
    wm, bm = _stack_tail(params['edge_conv2_msg'])

    epars = params['edge_conv2_edge']
    we = jnp.stack([epars[0][0][:C]] + [w for (w, b) in epars[1:]])
    be = jnp.stack([b for (w, b) in epars])
    ef1g = ef1 @ epars[0][0][C:]

    (wl1, bl1) = params['out_lcf1'][0]
    (wl2, bl2) = params['out_lcf2'][0]
    ef_pad = jnp.concatenate(
        [edge_features_1d,
         jnp.zeros((PADH - e_und, edge_features_1d.shape[1]), jnp.float32)], axis=0)
    eff = ef_pad @ wl1[C : C + 16] + ew_pad * wl1[C + 16][None, :]
    wh_head = wl1[:C]
    bh_head = jnp.stack([
        bl1,
        wl2[:, 0],
        wl2[:, 1],
        jnp.concatenate([bl2, jnp.zeros((C - 2,), jnp.float32)]),
    ])

    out = _e2_head(ga2, gb2, ef1g, eff, ew_pad,
                   wm, bm, we, be, wh_head, bh_head)
    return out[:e_und]


# speedup vs baseline: 1.3177x; 1.0730x over previous
import functools

import jax
import jax.numpy as jnp
from jax import lax
from jax.experimental import pallas as pl
from jax.experimental.pallas import tpu as pltpu
from jax.experimental.pallas import tpu_sc as plsc

C = 256
N_HIDDEN = 5
BE = 1280
KCH = 104
CW = 100
NW = 32
PADH = 16 * KCH * CW
PADT = 2 * PADH


def _leaky(x):
    return jnp.maximum(x, 0.01 * x)


def _dot(a, b, out_dtype=jnp.float32):
    return jax.lax.dot_general(a.astype(jnp.bfloat16), b.astype(jnp.bfloat16),
                               (((1,), (0,)), ((), ())),
                               preferred_element_type=out_dtype)


def _pack_bf16(a):
    return jax.lax.bitcast_convert_type(
        a.reshape(a.shape[0], C // 2, 2), jnp.float32)


def _sc_gather2(taba, tabb, idxa, idxb):
    mesh = plsc.VectorSubcoreMesh(core_axis_name="c", subcore_axis_name="s")
    dt = taba.dtype
    nc = taba.shape[1]

    @functools.partial(
        pl.kernel,
        mesh=mesh,
        out_type=(jax.ShapeDtypeStruct((PADT, nc), dt),
                  jax.ShapeDtypeStruct((PADT, nc), dt)),
        scratch_types=(
            [pltpu.VMEM((KCH, nc), dt)] * 4
            + [pltpu.VMEM((KCH,), jnp.int32)] * 8
            + [pltpu.SemaphoreType.DMA] * 16
        ),
    )
    def k(ta_h, tb_h, ia_h, ib_h, oa_h, ob_h,
          ra0, ra1, rb0, rb1,
          ja0, ja1, ja2, ja3, jb0, jb1, jb2, jb3,
          sia0, sia1, sia2, sia3, sib0, sib1, sib2, sib3,
          sga0, sga1, sgb0, sgb1, ssa0, ssa1, ssb0, ssb1):
        wid = lax.axis_index("s") * 2 + lax.axis_index("c")
        base = wid * CW

        ra = (ra0, ra1)
        rb = (rb0, rb1)
        ja = (ja0, ja1, ja2, ja3)
        jb = (jb0, jb1, jb2, jb3)
        sia = (sia0, sia1, sia2, sia3)
        sib = (sib0, sib1, sib2, sib3)
        sga = (sga0, sga1)
        sgb = (sgb0, sgb1)
        ssa = (ssa0, ssa1)
        ssb = (ssb0, ssb1)

        def fire_idx(j, q):
            off = (base + j) * KCH
            pltpu.async_copy(ia_h.at[pl.ds(off, KCH)], ja[q], sia[q])
            pltpu.async_copy(ib_h.at[pl.ds(off, KCH)], jb[q], sib[q])

        def gath(q, p):
            pltpu.make_async_copy(ia_h.at[pl.ds(0, KCH)], ja[q], sia[q]).wait()
            pltpu.make_async_copy(ib_h.at[pl.ds(0, KCH)], jb[q], sib[q]).wait()
            pltpu.async_copy(ta_h.at[ja[q]], ra[p], sga[p])
            pltpu.async_copy(tb_h.at[jb[q]], rb[p], sgb[p])

        def stor(j, q, p):
            off = (base + j) * KCH
            pltpu.make_async_copy(ta_h.at[ja[q]], ra[p], sga[p]).wait()
            pltpu.make_async_copy(tb_h.at[jb[q]], rb[p], sgb[p]).wait()
            pltpu.async_copy(ra[p], oa_h.at[pl.ds(off, KCH)], ssa[p])
            pltpu.async_copy(rb[p], ob_h.at[pl.ds(off, KCH)], ssb[p])

        def wait_stor(p):
            pltpu.make_async_copy(ra[p], oa_h.at[pl.ds(0, KCH)], ssa[p]).wait()
            pltpu.make_async_copy(rb[p], ob_h.at[pl.ds(0, KCH)], ssb[p]).wait()

        fire_idx(0, 0)
        fire_idx(1, 1)
        fire_idx(2, 2)
        fire_idx(3, 3)

        def body(jj, carry):
            for u in range(4):
                j = 4 * jj + u
                q = u
                p = u % 2

                if u >= 2:
                    wait_stor(p)
                else:
                    @pl.when(jj > 0)
                    def _():
                        wait_stor(p)

                gath(q, p)

                if u >= 1:
                    qp = u - 1
                    pp = (u - 1) % 2
                    stor(j - 1, qp, pp)
                    fire_j = j + 3

                    @pl.when(fire_j < CW)
                    def _():
                        fire_idx(fire_j, qp)
                else:
                    @pl.when(jj > 0)
                    def _():
                        stor(j - 1, 3, 1)

                        @pl.when(j + 3 < CW)
                        def _():
                            fire_idx(j + 3, 3)

            return carry

        lax.fori_loop(0, CW // 4, body, 0)
        stor(CW - 1, 3, 1)
        wait_stor(0)
        wait_stor(1)

    return k(taba, tabb, idxa, idxb)


def _mlp_tail(g, w_ref, b_ref):
    h = _leaky(g).astype(jnp.bfloat16)
    for i in range(N_HIDDEN - 1):
        z = _dot(h, w_ref[i]).astype(jnp.bfloat16)
        h = _leaky(z + b_ref[i : i + 1, :].astype(jnp.bfloat16))
    return _dot(h, w_ref[N_HIDDEN - 1]) + b_ref[N_HIDDEN - 1 : N_HIDDEN, :]


def _unpack(x):
    return x.astype(jnp.bfloat16)


def _mlp5n_kernel(ga_ref, gb_ref, ang_ref, w0a_ref, w_ref, b_ref, out_ref):
    g = (_unpack(ga_ref[...]) + _unpack(gb_ref[...])
         + (ang_ref[...] * w0a_ref[...]).astype(jnp.bfloat16))
    out_ref[...] = _mlp_tail(g, w_ref, b_ref).astype(jnp.bfloat16)


def _mlp5_pair_kernel(galo_ref, gblo_ref, gahi_ref, gbhi_ref, ew_ref,
                      w_ref, b_ref, out_ref):
    mlo = _mlp_tail(_unpack(galo_ref[...]) + _unpack(gblo_ref[...]), w_ref, b_ref)
    mhi = _mlp_tail(_unpack(gahi_ref[...]) + _unpack(gbhi_ref[...]), w_ref, b_ref)
    out_ref[...] = _leaky((mlo + mhi) * ew_ref[...]).astype(jnp.bfloat16)


def _e2_head_kernel(galo_ref, gblo_ref, gahi_ref, gbhi_ref, ef1g_ref, eff_ref,
                    ew_ref, wm_ref, bm_ref, we_ref, be_ref, wh_ref, bh_ref,
                    out_ref):
    ew = ew_ref[...]
    m = (_mlp_tail(_unpack(galo_ref[...]) + _unpack(gblo_ref[...]), wm_ref, bm_ref)
         + _mlp_tail(_unpack(gahi_ref[...]) + _unpack(gbhi_ref[...]), wm_ref, bm_ref))
    ef2pre = m * ew
    z = _dot(ef2pre, we_ref[0]) + ef1g_ref[...] + be_ref[0:1, :]
    h = _leaky(z).astype(jnp.bfloat16)
    for i in range(1, N_HIDDEN):
        zz = _dot(h, we_ref[i]).astype(jnp.bfloat16)
        h = _leaky(zz + be_ref[i : i + 1, :].astype(jnp.bfloat16))
    ef2 = _leaky(_dot(h, we_ref[N_HIDDEN]) + be_ref[N_HIDDEN : N_HIDDEN + 1, :])
    e1 = _dot(ef2, wh_ref[...]) + eff_ref[...] + bh_ref[0:1, :]
    s0 = jnp.sum(e1 * bh_ref[1:2, :], axis=1, keepdims=True) + bh_ref[3:4, 0:1]
    s1 = jnp.sum(e1 * bh_ref[2:3, :], axis=1, keepdims=True) + bh_ref[3:4, 1:2]
    s0 = jax.nn.sigmoid(s0)
    s1 = jax.nn.sigmoid(s1)
    mx = jnp.maximum(s0, s1)
    z0 = jnp.exp(s0 - mx)
    z1 = jnp.exp(s1 - mx)
    tot = z0 + z1
    out_ref[...] = jnp.concatenate([z0 / tot, z1 / tot], axis=1)


def _full_spec(shape):
    return pl.BlockSpec(shape, lambda i: tuple(0 for _ in shape))


def _row_spec(cols=C):
    return pl.BlockSpec((BE, cols), lambda i: (i, 0))


def _hi_spec(cols=C):
    off = PADH // BE
    return pl.BlockSpec((BE, cols), lambda i: (i + off, 0))


def _mlp5n(ga, gb, ang, w0a, wh, bh):
    grid = (PADT // BE,)
    return pl.pallas_call(
        _mlp5n_kernel,
        grid=grid,
        in_specs=[_row_spec(), _row_spec(), _row_spec(1),
                  _full_spec(w0a.shape), _full_spec(wh.shape), _full_spec(bh.shape)],
        out_specs=_row_spec(),
        out_shape=jax.ShapeDtypeStruct((PADT, C), jnp.bfloat16),
    )(ga, gb, ang, w0a, wh, bh)


def _mlp5_pair(ga, gb, ew, wh, bh):
    grid = (PADH // BE,)
    return pl.pallas_call(
        _mlp5_pair_kernel,
        grid=grid,
        in_specs=[_row_spec(), _row_spec(), _hi_spec(), _hi_spec(),
                  pl.BlockSpec((BE, 1), lambda i: (i, 0)),
                  _full_spec(wh.shape), _full_spec(bh.shape)],
        out_specs=_row_spec(),
        out_shape=jax.ShapeDtypeStruct((PADH, C), jnp.bfloat16),
    )(ga, gb, ga, gb, ew, wh, bh)


def _e2_head(ga, gb, ef1g, eff, ew, wm, bm, we, be, wh, bh):
    grid = (PADH // BE,)
    return pl.pallas_call(
        _e2_head_kernel,
        grid=grid,
        in_specs=[_row_spec(), _row_spec(), _hi_spec(), _hi_spec(),
                  _row_spec(), _row_spec(),
                  pl.BlockSpec((BE, 1), lambda i: (i, 0)),
                  _full_spec(wm.shape), _full_spec(bm.shape),
                  _full_spec(we.shape), _full_spec(be.shape),
                  _full_spec(wh.shape), _full_spec(bh.shape)],
        out_specs=pl.BlockSpec((BE, 2), lambda i: (i, 0)),
        out_shape=jax.ShapeDtypeStruct((PADH, 2), jnp.float32),
    )(ga, gb, ga, gb, ef1g, eff, ew, wm, bm, we, be, wh, bh)


def _stack_tail(pars):
    wh = jnp.stack([w for (w, b) in pars[1:]])
    bh = jnp.stack([b for (w, b) in pars[1:]])
    return wh, bh


def _pad_half(a, n):
    pad = jnp.zeros((PADH - n,) + a.shape[1:], a.dtype)
    return jnp.concatenate([a[:n], pad, a[n:], pad], axis=0)


def kernel(node_features, edge_features_1d, edge_index, angles, edge_weights, params):
    src = edge_index[0].astype(jnp.int32)
    dst = edge_index[1].astype(jnp.int32)
    n_nodes = node_features.shape[0]
    e_und = edge_weights.shape[0]

    src_pad = _pad_half(src, e_und)
    dst_pad = _pad_half(dst, e_und)
    half_mask = jnp.concatenate([
        jnp.zeros((e_und,), jnp.int32),
        jnp.ones((PADH - e_und,), jnp.int32),
    ])
    pad_mask = jnp.concatenate([half_mask, half_mask])
    dst_scat = jnp.where(pad_mask == 1, n_nodes, dst_pad)

    ang_pad = _pad_half(angles, e_und)
    ew_pad = jnp.concatenate(
        [edge_weights[:, None],
         jnp.zeros((PADH - e_und, 1), jnp.float32)], axis=0)

    cnt = jax.ops.segment_sum(jnp.ones((PADT,), jnp.float32), dst_scat,
                              num_segments=n_nodes + 1)[:n_nodes]
    inv = 1.0 / jnp.maximum(cnt, 1.0)

    def node_stage(x, pars):
        w0, b0 = pars[0]
        ta = x @ w0[:C] + b0[None, :]
        tb = x @ w0[C : 2 * C]
        ga, gb = _sc_gather2(ta, tb, src_pad, dst_pad)
        wh, bh = _stack_tail(pars)
        m = _mlp5n(ga, gb, ang_pad, w0[2 * C][None, :], wh, bh)
        s = jax.ops.segment_sum(m, dst_scat, num_segments=n_nodes + 1)[:n_nodes]
        return _leaky(s.astype(jnp.float32) * inv[:, None])

    x1 = node_stage(node_features, params['node_conv1_msg'])

    v0, c0 = params['edge_conv1_msg'][0]
    ta = x1 @ v0[:C] + c0[None, :]
    tb = x1 @ v0[C:]
    ga1, gb1 = _sc_gather2(ta, tb, src_pad, dst_pad)
    wh1, bh1 = _stack_tail(params['edge_conv1_msg'])
    ef1 = _mlp5_pair(ga1, gb1, ew_pad, wh1, bh1)

    x2 = node_stage(x1, params['node_conv2_msg'])

    u0, d0 = params['edge_conv2_msg'][0]
    ta2 = x2 @ u0[:C] + d0[None, :]
    tb2 = x2 @ u0[C:]
    ga2, gb2 = _sc_gather2(ta2, tb2, src_pad, dst_pad)
    wm, bm = _stack_tail(params['edge_conv2_msg'])

    epars = params['edge_conv2_edge']
    we = jnp.stack([epars[0][0][:C]] + [w for (w, b) in epars[1:]])
    be = jnp.stack([b for (w, b) in epars])
    ef1g = jnp.dot(ef1, epars[0][0][C:].astype(jnp.bfloat16),
                   preferred_element_type=jnp.float32).astype(jnp.bfloat16)

    (wl1, bl1) = params['out_lcf1'][0]
    (wl2, bl2) = params['out_lcf2'][0]
    ef_pad = jnp.concatenate(
        [edge_features_1d,
         jnp.zeros((PADH - e_und, edge_features_1d.shape[1]), jnp.float32)], axis=0)
    eff = (ef_pad @ wl1[C : C + 16] + ew_pad * wl1[C + 16][None, :]).astype(jnp.bfloat16)
    wh_head = wl1[:C]
    bh_head = jnp.stack([
        bl1,
        wl2[:, 0],
        wl2[:, 1],
        jnp.concatenate([bl2, jnp.zeros((C - 2,), jnp.float32)]),
    ])

    out = _e2_head(ga2, gb2, ef1g, eff, ew_pad,
                   wm, bm, we, be, wh_head, bh_head)
    return out[:e_und]


# speedup vs baseline: 1.7064x; 1.2950x over previous
import functools

import jax
import jax.numpy as jnp
from jax import lax
from jax.experimental import pallas as pl
from jax.experimental.pallas import tpu as pltpu
from jax.experimental.pallas import tpu_sc as plsc

C = 256
N_HIDDEN = 5
BE = 1280
KCH = 56
CW = 180
NW = 32
NP = 4
PADH = 16 * KCH * CW
PADT = 2 * PADH


def _leaky(x):
    return jnp.maximum(x, 0.01 * x)


def _dot(a, b, out_dtype=jnp.float32):
    return jax.lax.dot_general(a.astype(jnp.bfloat16), b.astype(jnp.bfloat16),
                               (((1,), (0,)), ((), ())),
                               preferred_element_type=out_dtype)


def _pack_bf16(a):
    return jax.lax.bitcast_convert_type(
        a.reshape(a.shape[0], C // 2, 2), jnp.float32)


def _sc_gather2(taba, tabb, idxa, idxb):
    mesh = plsc.VectorSubcoreMesh(core_axis_name="c", subcore_axis_name="s")
    dt = taba.dtype
    nc = taba.shape[1]

    @functools.partial(
        pl.kernel,
        mesh=mesh,
        out_type=(jax.ShapeDtypeStruct((PADT, nc), dt),
                  jax.ShapeDtypeStruct((PADT, nc), dt)),
        scratch_types=(
            [pltpu.VMEM((KCH, nc), dt)] * (2 * NP)
            + [pltpu.VMEM((KCH,), jnp.int32)] * (2 * NP)
            + [pltpu.SemaphoreType.DMA] * (6 * NP)
        ),
    )
    def k(ta_h, tb_h, ia_h, ib_h, oa_h, ob_h, *scr):
        ra = scr[0:NP]
        rb = scr[NP:2 * NP]
        ja = scr[2 * NP:3 * NP]
        jb = scr[3 * NP:4 * NP]
        sia = scr[4 * NP:5 * NP]
        sib = scr[5 * NP:6 * NP]
        sga = scr[6 * NP:7 * NP]
        sgb = scr[7 * NP:8 * NP]
        ssa = scr[8 * NP:9 * NP]
        ssb = scr[9 * NP:10 * NP]
        wid = lax.axis_index("s") * 2 + lax.axis_index("c")
        base = wid * CW

        def fire_idx(j, q):
            off = (base + j) * KCH
            pltpu.async_copy(ia_h.at[pl.ds(off, KCH)], ja[q], sia[q])
            pltpu.async_copy(ib_h.at[pl.ds(off, KCH)], jb[q], sib[q])

        def gath(p):
            pltpu.make_async_copy(ia_h.at[pl.ds(0, KCH)], ja[p], sia[p]).wait()
            pltpu.make_async_copy(ib_h.at[pl.ds(0, KCH)], jb[p], sib[p]).wait()
            pltpu.async_copy(ta_h.at[ja[p]], ra[p], sga[p])
            pltpu.async_copy(tb_h.at[jb[p]], rb[p], sgb[p])

        def stor(j, p):
            off = (base + j) * KCH
            pltpu.make_async_copy(ta_h.at[ja[p]], ra[p], sga[p]).wait()
            pltpu.make_async_copy(tb_h.at[jb[p]], rb[p], sgb[p]).wait()
            pltpu.async_copy(ra[p], oa_h.at[pl.ds(off, KCH)], ssa[p])
            pltpu.async_copy(rb[p], ob_h.at[pl.ds(off, KCH)], ssb[p])

        def wait_stor(p):
            pltpu.make_async_copy(ra[p], oa_h.at[pl.ds(0, KCH)], ssa[p]).wait()
            pltpu.make_async_copy(rb[p], ob_h.at[pl.ds(0, KCH)], ssb[p]).wait()

        for q in range(NP):
            fire_idx(q, q)

        def body(jj, carry):
            for u in range(NP):
                j = NP * jj + u

                if u == 0:
                    @pl.when(jj > 0)
                    def _():
                        wait_stor(0)
                else:
                    @pl.when(jj > 0)
                    def _():
                        wait_stor(u)

                gath(u)

                if u >= 1:
                    stor(j - 1, u - 1)

                    @pl.when(j + NP - 1 < CW)
                    def _():
                        fire_idx(j + NP - 1, u - 1)
                else:
                    @pl.when(jj > 0)
                    def _():
                        stor(j - 1, NP - 1)

                        @pl.when(j + NP - 1 < CW)
                        def _():
                            fire_idx(j + NP - 1, NP - 1)

            return carry

        lax.fori_loop(0, CW // NP, body, 0)
        stor(CW - 1, NP - 1)
        for p in range(NP):
            wait_stor(p)

    return k(taba, tabb, idxa, idxb)


def _mlp_tail(g, w_ref, b_ref):
    h = _leaky(g).astype(jnp.bfloat16)
    for i in range(N_HIDDEN - 1):
        z = _dot(h, w_ref[i]).astype(jnp.bfloat16)
        h = _leaky(z + b_ref[i : i + 1, :].astype(jnp.bfloat16))
    return _dot(h, w_ref[N_HIDDEN - 1]) + b_ref[N_HIDDEN - 1 : N_HIDDEN, :]


def _unpack(x):
    return x.astype(jnp.bfloat16)


def _mlp5n_kernel(ga_ref, gb_ref, ang_ref, w0a_ref, w_ref, b_ref, out_ref):
    g = (_unpack(ga_ref[...]) + _unpack(gb_ref[...])
         + (ang_ref[...] * w0a_ref[...]).astype(jnp.bfloat16))
    out_ref[...] = _mlp_tail(g, w_ref, b_ref).astype(jnp.bfloat16)


def _mlp5_pair_kernel(galo_ref, gblo_ref, gahi_ref, gbhi_ref, ew_ref,
                      w_ref, b_ref, out_ref):
    mlo = _mlp_tail(_unpack(galo_ref[...]) + _unpack(gblo_ref[...]), w_ref, b_ref)
    mhi = _mlp_tail(_unpack(gahi_ref[...]) + _unpack(gbhi_ref[...]), w_ref, b_ref)
    out_ref[...] = _leaky((mlo + mhi) * ew_ref[...]).astype(jnp.bfloat16)


def _e2_head_kernel(galo_ref, gblo_ref, gahi_ref, gbhi_ref, ef1g_ref, eff_ref,
                    ew_ref, wm_ref, bm_ref, we_ref, be_ref, wh_ref, bh_ref,
                    out_ref):
    ew = ew_ref[...]
    m = (_mlp_tail(_unpack(galo_ref[...]) + _unpack(gblo_ref[...]), wm_ref, bm_ref)
         + _mlp_tail(_unpack(gahi_ref[...]) + _unpack(gbhi_ref[...]), wm_ref, bm_ref))
    ef2pre = m * ew
    z = _dot(ef2pre, we_ref[0]) + ef1g_ref[...] + be_ref[0:1, :]
    h = _leaky(z).astype(jnp.bfloat16)
    for i in range(1, N_HIDDEN):
        zz = _dot(h, we_ref[i]).astype(jnp.bfloat16)
        h = _leaky(zz + be_ref[i : i + 1, :].astype(jnp.bfloat16))
    ef2 = _leaky(_dot(h, we_ref[N_HIDDEN]) + be_ref[N_HIDDEN : N_HIDDEN + 1, :])
    e1 = _dot(ef2, wh_ref[...]) + eff_ref[...] + bh_ref[0:1, :]
    s0 = jnp.sum(e1 * bh_ref[1:2, :], axis=1, keepdims=True) + bh_ref[3:4, 0:1]
    s1 = jnp.sum(e1 * bh_ref[2:3, :], axis=1, keepdims=True) + bh_ref[3:4, 1:2]
    s0 = jax.nn.sigmoid(s0)
    s1 = jax.nn.sigmoid(s1)
    mx = jnp.maximum(s0, s1)
    z0 = jnp.exp(s0 - mx)
    z1 = jnp.exp(s1 - mx)
    tot = z0 + z1
    out_ref[...] = jnp.concatenate([z0 / tot, z1 / tot], axis=1)


def _full_spec(shape):
    return pl.BlockSpec(shape, lambda i: tuple(0 for _ in shape))


def _row_spec(cols=C):
    return pl.BlockSpec((BE, cols), lambda i: (i, 0))


def _hi_spec(cols=C):
    off = PADH // BE
    return pl.BlockSpec((BE, cols), lambda i: (i + off, 0))


def _mlp5n(ga, gb, ang, w0a, wh, bh):
    grid = (PADT // BE,)
    return pl.pallas_call(
        _mlp5n_kernel,
        grid=grid,
        in_specs=[_row_spec(), _row_spec(), _row_spec(1),
                  _full_spec(w0a.shape), _full_spec(wh.shape), _full_spec(bh.shape)],
        out_specs=_row_spec(),
        out_shape=jax.ShapeDtypeStruct((PADT, C), jnp.bfloat16),
    )(ga, gb, ang, w0a, wh, bh)


def _mlp5_pair(ga, gb, ew, wh, bh):
    grid = (PADH // BE,)
    return pl.pallas_call(
        _mlp5_pair_kernel,
        grid=grid,
        in_specs=[_row_spec(), _row_spec(), _hi_spec(), _hi_spec(),
                  pl.BlockSpec((BE, 1), lambda i: (i, 0)),
                  _full_spec(wh.shape), _full_spec(bh.shape)],
        out_specs=_row_spec(),
        out_shape=jax.ShapeDtypeStruct((PADH, C), jnp.bfloat16),
    )(ga, gb, ga, gb, ew, wh, bh)


def _e2_head(ga, gb, ef1g, eff, ew, wm, bm, we, be, wh, bh):
    grid = (PADH // BE,)
    return pl.pallas_call(
        _e2_head_kernel,
        grid=grid,
        in_specs=[_row_spec(), _row_spec(), _hi_spec(), _hi_spec(),
                  _row_spec(), _row_spec(),
                  pl.BlockSpec((BE, 1), lambda i: (i, 0)),
                  _full_spec(wm.shape), _full_spec(bm.shape),
                  _full_spec(we.shape), _full_spec(be.shape),
                  _full_spec(wh.shape), _full_spec(bh.shape)],
        out_specs=pl.BlockSpec((BE, 2), lambda i: (i, 0)),
        out_shape=jax.ShapeDtypeStruct((PADH, 2), jnp.float32),
    )(ga, gb, ga, gb, ef1g, eff, ew, wm, bm, we, be, wh, bh)


def _stack_tail(pars):
    wh = jnp.stack([w for (w, b) in pars[1:]])
    bh = jnp.stack([b for (w, b) in pars[1:]])
    return wh, bh


def _pad_half(a, n):
    pad = jnp.zeros((PADH - n,) + a.shape[1:], a.dtype)
    return jnp.concatenate([a[:n], pad, a[n:], pad], axis=0)


def kernel(node_features, edge_features_1d, edge_index, angles, edge_weights, params):
    src = edge_index[0].astype(jnp.int32)
    dst = edge_index[1].astype(jnp.int32)
    n_nodes = node_features.shape[0]
    e_und = edge_weights.shape[0]

    src_pad = _pad_half(src, e_und)
    dst_pad = _pad_half(dst, e_und)
    half_mask = jnp.concatenate([
        jnp.zeros((e_und,), jnp.int32),
        jnp.ones((PADH - e_und,), jnp.int32),
    ])
    pad_mask = jnp.concatenate([half_mask, half_mask])
    dst_scat = jnp.where(pad_mask == 1, n_nodes, dst_pad)

    ang_pad = _pad_half(angles, e_und)
    ew_pad = jnp.concatenate(
        [edge_weights[:, None],
         jnp.zeros((PADH - e_und, 1), jnp.float32)], axis=0)

    cnt = jax.ops.segment_sum(jnp.ones((PADT,), jnp.float32), dst_scat,
                              num_segments=n_nodes + 1)[:n_nodes]
    inv = 1.0 / jnp.maximum(cnt, 1.0)

    def node_stage(x, pars):
        w0, b0 = pars[0]
        ta = x @ w0[:C] + b0[None, :]
        tb = x @ w0[C : 2 * C]
        ga, gb = _sc_gather2(ta, tb, src_pad, dst_pad)
        wh, bh = _stack_tail(pars)
        m = _mlp5n(ga, gb, ang_pad, w0[2 * C][None, :], wh, bh)
        s = jax.ops.segment_sum(m, dst_scat, num_segments=n_nodes + 1)[:n_nodes]
        return _leaky(s.astype(jnp.float32) * inv[:, None])

    x1 = node_stage(node_features, params['node_conv1_msg'])

    v0, c0 = params['edge_conv1_msg'][0]
    ta = x1 @ v0[:C] + c0[None, :]
    tb = x1 @ v0[C:]
    ga1, gb1 = _sc_gather2(ta, tb, src_pad, dst_pad)
    wh1, bh1 = _stack_tail(params['edge_conv1_msg'])
    ef1 = _mlp5_pair(ga1, gb1, ew_pad, wh1, bh1)

    x2 = node_stage(x1, params['node_conv2_msg'])

    u0, d0 = params['edge_conv2_msg'][0]
    ta2 = x2 @ u0[:C] + d0[None, :]
    tb2 = x2 @ u0[C:]
    ga2, gb2 = _sc_gather2(ta2, tb2, src_pad, dst_pad)
    wm, bm = _stack_tail(params['edge_conv2_msg'])

    epars = params['edge_conv2_edge']
    we = jnp.stack([epars[0][0][:C]] + [w for (w, b) in epars[1:]])
    be = jnp.stack([b for (w, b) in epars])
    ef1g = jnp.dot(ef1, epars[0][0][C:].astype(jnp.bfloat16),
                   preferred_element_type=jnp.float32).astype(jnp.bfloat16)

    (wl1, bl1) = params['out_lcf1'][0]
    (wl2, bl2) = params['out_lcf2'][0]
    ef_pad = jnp.concatenate(
        [edge_features_1d,
         jnp.zeros((PADH - e_und, edge_features_1d.shape[1]), jnp.float32)], axis=0)
    eff = (ef_pad @ wl1[C : C + 16] + ew_pad * wl1[C + 16][None, :]).astype(jnp.bfloat16)
    wh_head = wl1[:C]
    bh_head = jnp.stack([
        bl1,
        wl2[:, 0],
        wl2[:, 1],
        jnp.concatenate([bl2, jnp.zeros((C - 2,), jnp.float32)]),
    ])

    out = _e2_head(ga2, gb2, ef1g, eff, ew_pad,
                   wm, bm, we, be, wh_head, bh_head)
    return out[:e_und]
